# Initial kernel scaffold; baseline (speedup 1.0000x reference)
import jax
import jax.numpy as jnp
from jax.experimental import pallas as pl


def kernel(x, edge_index, edge_attr, W1, b1, W2, b2, W3, b3, g1, bt1, m1, v1, g2, bt2, m2, v2, g3, bt3, m3, v3, eW1, eb1, eW2, eb2, eW3, eb3):
    raise NotImplementedError("write your pallas kernel here")


# speedup vs baseline: 11.2256x; 11.2256x over previous
import jax
import jax.numpy as jnp
from jax import lax
from jax.experimental import pallas as pl
from jax.experimental.pallas import tpu as pltpu
from jax.experimental.pallas import tpu_sc as plsc

_N = 10000
_E = 320000
_D = 128
_H = 128
_DE = 16
_EPS = 1e-5

_NC = 2
_NS = 16
_NW = _NC * _NS
_EPW = _E // _NW
_EPT = _E // _NS
_ZS = 624
_ZR = 312
_CDEG = 2000
_CEG = 2000


def _deg_body(dst_hbm, out_hbm, idx_v, ones_v, zbuf_v, acc_sh):
    cid = lax.axis_index("c")
    sid = lax.axis_index("s")
    wid = sid * _NC + cid
    for j in range(_CDEG // 16):
        ones_v[pl.ds(16 * j, 16)] = jnp.ones((16,), jnp.float32)
    for j in range(_ZS // 16):
        zbuf_v[pl.ds(16 * j, 16)] = jnp.zeros((16,), jnp.float32)
    pltpu.sync_copy(zbuf_v, acc_sh.at[pl.ds(sid * _ZS, _ZS)])

    @pl.when(sid == 0)
    def _():
        pltpu.sync_copy(zbuf_v.at[pl.ds(0, 16)], acc_sh.at[pl.ds(_NS * _ZS, 16)])

    plsc.subcore_barrier()

    def chunk(k, c):
        base = wid * _EPW + k * _CDEG
        pltpu.sync_copy(dst_hbm.at[pl.ds(base, _CDEG)], idx_v)
        pltpu.sync_copy(ones_v, acc_sh.at[idx_v], add=True)
        return c

    lax.fori_loop(0, _EPW // _CDEG, chunk, 0)
    plsc.subcore_barrier()
    pltpu.sync_copy(acc_sh.at[pl.ds(sid * _ZS, _ZS)], zbuf_v)
    pltpu.sync_copy(zbuf_v, out_hbm.at[pl.ds(cid * _N + sid * _ZS, _ZS)])

    @pl.when(sid == 0)
    def _():
        pltpu.sync_copy(acc_sh.at[pl.ds(_NS * _ZS, 16)], zbuf_v.at[pl.ds(0, 16)])
        pltpu.sync_copy(zbuf_v.at[pl.ds(0, 16)],
                        out_hbm.at[pl.ds(cid * _N + _NS * _ZS, 16)])


def _make_prop_body(hdh, chp):
    def body(lo_hbm, hi_hbm, src_hbm, dst_hbm, out_lo_hbm, out_hi_hbm,
             sidx_v, didx_v, rows_v, zrow_v, acc_sh, sem):
        cid = lax.axis_index("c")
        sid = lax.axis_index("s")

        def zfill(i, c):
            for j in range(hdh // 16):
                zrow_v[i, pl.ds(16 * j, 16)] = jnp.zeros((16,), jnp.float32)
            return c

        lax.fori_loop(0, _ZR, zfill, 0)

        def zcopy(r, c):
            pltpu.sync_copy(zrow_v, acc_sh.at[pl.ds(sid * _ZS + r * _ZR, _ZR)])
            return c

        lax.fori_loop(0, _ZS // _ZR, zcopy, 0)

        @pl.when(sid == 0)
        def _():
            pltpu.sync_copy(zrow_v.at[pl.ds(0, 16)],
                            acc_sh.at[pl.ds(_NS * _ZS, 16)])

        plsc.subcore_barrier()

        def chunk(k, c):
            base = sid * _EPT + k * chp
            pltpu.sync_copy(src_hbm.at[pl.ds(base, chp)], sidx_v)
            pltpu.sync_copy(dst_hbm.at[pl.ds(base, chp)], didx_v)

            @pl.when(cid == 0)
            def _():
                pltpu.async_copy(lo_hbm.at[sidx_v], rows_v, sem).wait()

            @pl.when(cid == 1)
            def _():
                pltpu.async_copy(hi_hbm.at[sidx_v], rows_v, sem).wait()

            pltpu.sync_copy(rows_v, acc_sh.at[didx_v], add=True)
            return c

        lax.fori_loop(0, _EPT // chp, chunk, 0)
        plsc.subcore_barrier()

        def make_rdout(out_hbm):
            def rdout(r, c):
                pltpu.sync_copy(acc_sh.at[pl.ds(sid * _ZS + r * _ZR, _ZR)],
                                zrow_v)
                pltpu.sync_copy(
                    zrow_v, out_hbm.at[pl.ds(sid * _ZS + r * _ZR, _ZR)])
                return c
            return rdout

        def rem_out(out_hbm):
            pltpu.sync_copy(acc_sh.at[pl.ds(_NS * _ZS, 16)],
                            zrow_v.at[pl.ds(0, 16)])
            pltpu.sync_copy(zrow_v.at[pl.ds(0, 16)],
                            out_hbm.at[pl.ds(_NS * _ZS, 16)])

        @pl.when(cid == 0)
        def _():
            lax.fori_loop(0, _ZS // _ZR, make_rdout(out_lo_hbm), 0)

            @pl.when(sid == 0)
            def _():
                rem_out(out_lo_hbm)

        @pl.when(cid == 1)
        def _():
            lax.fori_loop(0, _ZS // _ZR, make_rdout(out_hi_hbm), 0)

            @pl.when(sid == 0)
            def _():
                rem_out(out_hi_hbm)

    return body


def _egather_body(h3_hbm, src_hbm, dst_hbm, sf_hbm, df_hbm, idx_v, rows_v, sem):
    cid = lax.axis_index("c")
    sid = lax.axis_index("s")
    wid = sid * _NC + cid

    def chunk(k, c):
        base = wid * _EPW + k * _CEG
        pltpu.sync_copy(src_hbm.at[pl.ds(base, _CEG)], idx_v)
        pltpu.async_copy(h3_hbm.at[idx_v], rows_v, sem).wait()
        pltpu.sync_copy(rows_v, sf_hbm.at[pl.ds(base, _CEG)])
        pltpu.sync_copy(dst_hbm.at[pl.ds(base, _CEG)], idx_v)
        pltpu.async_copy(h3_hbm.at[idx_v], rows_v, sem).wait()
        pltpu.sync_copy(rows_v, df_hbm.at[pl.ds(base, _CEG)])
        return c

    lax.fori_loop(0, _EPW // _CEG, chunk, 0)


_BN = 1000
_BE = 4000


def _pre_body(x_ref, dis_ref, w_ref, olo_ref, ohi_ref):
    xb = jnp.nan_to_num(x_ref[...])
    res = (jnp.dot(xb, w_ref[...], preferred_element_type=jnp.float32)
           * dis_ref[...])
    hh = res.shape[1] // 2
    olo_ref[...] = res[:, :hh]
    ohi_ref[...] = res[:, hh:]


def _mid_body(slo_ref, shi_ref, plo_ref, phi_ref, dis_ref, b_ref, g_ref,
              bt_ref, m_ref, v_ref, w_ref, olo_ref, ohi_ref):
    dis = dis_ref[...]
    y = jnp.concatenate(
        [slo_ref[...] + plo_ref[...], shi_ref[...] + phi_ref[...]], axis=1)
    y = y * dis + b_ref[...]
    t = (y - m_ref[...]) * lax.rsqrt(v_ref[...] + _EPS) * g_ref[...] + bt_ref[...]
    t = jnp.maximum(t, 0.0)
    res = (jnp.dot(t, w_ref[...], preferred_element_type=jnp.float32) * dis)
    hh = res.shape[1] // 2
    olo_ref[...] = res[:, :hh]
    ohi_ref[...] = res[:, hh:]


def _fin_body(slo_ref, shi_ref, plo_ref, phi_ref, dis_ref, b_ref, g_ref,
              bt_ref, m_ref, v_ref, o_ref):
    dis = dis_ref[...]
    y = jnp.concatenate(
        [slo_ref[...] + plo_ref[...], shi_ref[...] + phi_ref[...]], axis=1)
    y = y * dis + b_ref[...]
    t = (y - m_ref[...]) * lax.rsqrt(v_ref[...] + _EPS) * g_ref[...] + bt_ref[...]
    o_ref[...] = jnp.maximum(t, 0.0)


def _mlp_body(sf_ref, df_ref, ea_ref, w1_ref, b1_ref, w2_ref, b2_ref,
              w3_ref, b3_ref, o_ref):
    ea = jnp.nan_to_num(ea_ref[...])
    z = (jnp.dot(sf_ref[...], w1_ref[0:32, :], preferred_element_type=jnp.float32)
         + jnp.dot(df_ref[...], w1_ref[32:64, :], preferred_element_type=jnp.float32)
         + jnp.dot(ea, w1_ref[64:80, :], preferred_element_type=jnp.float32)
         + b1_ref[...])
    z = jnp.maximum(z, 0.0)
    z = jnp.maximum(jnp.dot(z, w2_ref[...], preferred_element_type=jnp.float32)
                    + b2_ref[...], 0.0)
    o_ref[...] = jnp.nan_to_num(
        jnp.dot(z, w3_ref[...], preferred_element_type=jnp.float32) + b3_ref[...])


def _row_spec(hd):
    return pl.BlockSpec((_BN, hd), lambda i: (i, 0))


def _full_spec(r, c):
    return pl.BlockSpec((r, c), lambda i: (0, 0))


def kernel(x, edge_index, edge_attr, W1, b1, W2, b2, W3, b3,
           g1, bt1, m1, v1, g2, bt2, m2, v2, g3, bt3, m3, v3,
           eW1, eb1, eW2, eb2, eW3, eb3):
    src = edge_index[0]
    dst = edge_index[1]
    mesh = plsc.VectorSubcoreMesh(core_axis_name="c", subcore_axis_name="s")

    degp = pl.kernel(
        _deg_body,
        out_type=jax.ShapeDtypeStruct((_NC * _N,), jnp.float32),
        mesh=mesh,
        compiler_params=pltpu.CompilerParams(use_tc_tiling_on_sc=False),
        scratch_types=[
            pltpu.VMEM((_CDEG,), jnp.int32),
            pltpu.VMEM((_CDEG,), jnp.float32),
            pltpu.VMEM((_ZS,), jnp.float32),
            pltpu.VMEM_SHARED((_N,), jnp.float32),
        ],
    )(dst)
    dis_col = lax.rsqrt(degp[:_N] + degp[_N:] + 1.0)[:, None]

    def prop(hp_lo, hp_hi, hdh, chp):
        return pl.kernel(
            _make_prop_body(hdh, chp),
            out_type=[jax.ShapeDtypeStruct((_N, hdh), jnp.float32),
                      jax.ShapeDtypeStruct((_N, hdh), jnp.float32)],
            mesh=mesh,
            compiler_params=pltpu.CompilerParams(use_tc_tiling_on_sc=False),
            scratch_types=[
                pltpu.VMEM((chp,), jnp.int32),
                pltpu.VMEM((chp,), jnp.int32),
                pltpu.VMEM((chp, hdh), jnp.float32),
                pltpu.VMEM((_ZR, hdh), jnp.float32),
                pltpu.VMEM_SHARED((_N, hdh), jnp.float32),
                pltpu.SemaphoreType.DMA,
            ],
        )(hp_lo, hp_hi, src, dst)

    def half_outs(hd):
        return ([_row_spec(hd // 2), _row_spec(hd // 2)],
                [jax.ShapeDtypeStruct((_N, hd // 2), jnp.float32),
                 jax.ShapeDtypeStruct((_N, hd // 2), jnp.float32)])

    o_specs, o_shapes = half_outs(_H)
    p1_lo, p1_hi = pl.pallas_call(
        _pre_body,
        grid=(_N // _BN,),
        in_specs=[_row_spec(_D), _row_spec(1), _full_spec(_D, _H)],
        out_specs=o_specs,
        out_shape=o_shapes,
    )(x, dis_col, W1)

    s1_lo, s1_hi = prop(p1_lo, p1_hi, _H // 2, 1000)

    def mid(s_lo, s_hi, p_lo, p_hi, b, g, bt, m, v, W, hin, hout):
        o_specs, o_shapes = half_outs(hout)
        return pl.pallas_call(
            _mid_body,
            grid=(_N // _BN,),
            in_specs=[_row_spec(hin // 2), _row_spec(hin // 2),
                      _row_spec(hin // 2), _row_spec(hin // 2),
                      _row_spec(1),
                      _full_spec(1, hin), _full_spec(1, hin),
                      _full_spec(1, hin), _full_spec(1, hin),
                      _full_spec(1, hin), _full_spec(hin, hout)],
            out_specs=o_specs,
            out_shape=o_shapes,
        )(s_lo, s_hi, p_lo, p_hi, dis_col, b.reshape(1, -1), g.reshape(1, -1),
          bt.reshape(1, -1), m.reshape(1, -1), v.reshape(1, -1), W)

    p2_lo, p2_hi = mid(s1_lo, s1_hi, p1_lo, p1_hi, b1, g1, bt1, m1, v1,
                       W2, _H, _H // 2)
    s2_lo, s2_hi = prop(p2_lo, p2_hi, _H // 4, 2000)
    p3_lo, p3_hi = mid(s2_lo, s2_hi, p2_lo, p2_hi, b2, g2, bt2, m2, v2,
                       W3, _H // 2, _H // 4)
    s3_lo, s3_hi = prop(p3_lo, p3_hi, _H // 8, 4000)

    h3 = pl.pallas_call(
        _fin_body,
        grid=(_N // _BN,),
        in_specs=[_row_spec(_H // 8), _row_spec(_H // 8),
                  _row_spec(_H // 8), _row_spec(_H // 8),
                  _row_spec(1),
                  _full_spec(1, _H // 4), _full_spec(1, _H // 4),
                  _full_spec(1, _H // 4), _full_spec(1, _H // 4),
                  _full_spec(1, _H // 4)],
        out_specs=_row_spec(_H // 4),
        out_shape=jax.ShapeDtypeStruct((_N, _H // 4), jnp.float32),
    )(s3_lo, s3_hi, p3_lo, p3_hi, dis_col, b3.reshape(1, -1),
      g3.reshape(1, -1), bt3.reshape(1, -1), m3.reshape(1, -1),
      v3.reshape(1, -1))

    sf, df = pl.kernel(
        _egather_body,
        out_type=[jax.ShapeDtypeStruct((_E, _H // 4), jnp.float32),
                  jax.ShapeDtypeStruct((_E, _H // 4), jnp.float32)],
        mesh=mesh,
        compiler_params=pltpu.CompilerParams(use_tc_tiling_on_sc=False),
        scratch_types=[
            pltpu.VMEM((_CEG,), jnp.int32),
            pltpu.VMEM((_CEG, _H // 4), jnp.float32),
            pltpu.SemaphoreType.DMA,
        ],
    )(h3, src, dst)

    er = pl.BlockSpec((_BE, _H // 4), lambda i: (i, 0))
    out = pl.pallas_call(
        _mlp_body,
        grid=(_E // _BE,),
        in_specs=[er, er, pl.BlockSpec((_BE, _DE), lambda i: (i, 0)),
                  _full_spec(2 * (_H // 4) + _DE, _H // 2),
                  _full_spec(1, _H // 2),
                  _full_spec(_H // 2, _H // 4), _full_spec(1, _H // 4),
                  _full_spec(_H // 4, 2), _full_spec(1, 2)],
        out_specs=pl.BlockSpec((_BE, 2), lambda i: (i, 0)),
        out_shape=jax.ShapeDtypeStruct((_E, 2), jnp.float32),
    )(sf, df, edge_attr, eW1, eb1.reshape(1, -1), eW2, eb2.reshape(1, -1),
      eW3, eb3.reshape(1, -1))
    return out


# speedup vs baseline: 11.5214x; 1.0263x over previous
import jax
import jax.numpy as jnp
from jax import lax
from jax.experimental import pallas as pl
from jax.experimental.pallas import tpu as pltpu
from jax.experimental.pallas import tpu_sc as plsc

_N = 10000
_E = 320000
_D = 128
_H = 128
_DE = 16
_EPS = 1e-5

_NC = 2
_NS = 16
_NW = _NC * _NS
_EPW = _E // _NW
_EPT = _E // _NS
_ZS = 624
_ZR = 104
_CDEG = 2000
_CEG = 1000


def _deg_body(dst_hbm, out_hbm, idx_v, ones_v, zbuf_v, acc_sh):
    cid = lax.axis_index("c")
    sid = lax.axis_index("s")
    wid = sid * _NC + cid
    for j in range(_CDEG // 16):
        ones_v[pl.ds(16 * j, 16)] = jnp.ones((16,), jnp.float32)
    for j in range(_ZS // 16):
        zbuf_v[pl.ds(16 * j, 16)] = jnp.zeros((16,), jnp.float32)
    pltpu.sync_copy(zbuf_v, acc_sh.at[pl.ds(sid * _ZS, _ZS)])

    @pl.when(sid == 0)
    def _():
        pltpu.sync_copy(zbuf_v.at[pl.ds(0, 16)], acc_sh.at[pl.ds(_NS * _ZS, 16)])

    plsc.subcore_barrier()

    def chunk(k, c):
        base = wid * _EPW + k * _CDEG
        pltpu.sync_copy(dst_hbm.at[pl.ds(base, _CDEG)], idx_v)
        pltpu.sync_copy(ones_v, acc_sh.at[idx_v], add=True)
        return c

    lax.fori_loop(0, _EPW // _CDEG, chunk, 0)
    plsc.subcore_barrier()
    pltpu.sync_copy(acc_sh.at[pl.ds(sid * _ZS, _ZS)], zbuf_v)
    pltpu.sync_copy(zbuf_v, out_hbm.at[pl.ds(cid * _N + sid * _ZS, _ZS)])

    @pl.when(sid == 0)
    def _():
        pltpu.sync_copy(acc_sh.at[pl.ds(_NS * _ZS, 16)], zbuf_v.at[pl.ds(0, 16)])
        pltpu.sync_copy(zbuf_v.at[pl.ds(0, 16)],
                        out_hbm.at[pl.ds(cid * _N + _NS * _ZS, 16)])


def _make_prop_body(hdh, chp):
    def body(lo_hbm, hi_hbm, src_hbm, dst_hbm, out_lo_hbm, out_hi_hbm,
             sidx_a, didx_a, rows_a, sidx_b, didx_b, rows_b,
             zrow_v, acc_sh, sem_a, sem_b):
        cid = lax.axis_index("c")
        sid = lax.axis_index("s")

        def zfill(i, c):
            for j in range(hdh // 16):
                zrow_v[i, pl.ds(16 * j, 16)] = jnp.zeros((16,), jnp.float32)
            return c

        lax.fori_loop(0, _ZR, zfill, 0)

        def zcopy(r, c):
            pltpu.sync_copy(zrow_v, acc_sh.at[pl.ds(sid * _ZS + r * _ZR, _ZR)])
            return c

        lax.fori_loop(0, _ZS // _ZR, zcopy, 0)

        @pl.when(sid == 0)
        def _():
            pltpu.sync_copy(zrow_v.at[pl.ds(0, 16)],
                            acc_sh.at[pl.ds(_NS * _ZS, 16)])

        plsc.subcore_barrier()

        bufs = ((sidx_a, didx_a, rows_a, sem_a), (sidx_b, didx_b, rows_b, sem_b))

        def chunk(s, c):
            def do(sidx_v, didx_v, rows_v, sem):
                @pl.when(s >= 2)
                def _():
                    pltpu.make_async_copy(rows_v, acc_sh.at[didx_v], sem).wait()

                base = sid * _EPT + s * chp
                pltpu.sync_copy(src_hbm.at[pl.ds(base, chp)], sidx_v)
                pltpu.sync_copy(dst_hbm.at[pl.ds(base, chp)], didx_v)

                @pl.when(cid == 0)
                def _():
                    pltpu.sync_copy(lo_hbm.at[sidx_v], rows_v)

                @pl.when(cid == 1)
                def _():
                    pltpu.sync_copy(hi_hbm.at[sidx_v], rows_v)

                pltpu.async_copy(rows_v, acc_sh.at[didx_v], sem, add=True)

            @pl.when(s % 2 == 0)
            def _():
                do(*bufs[0])

            @pl.when(s % 2 == 1)
            def _():
                do(*bufs[1])

            return c

        lax.fori_loop(0, _EPT // chp, chunk, 0)
        pltpu.make_async_copy(rows_a, acc_sh.at[didx_a], sem_a).wait()
        pltpu.make_async_copy(rows_b, acc_sh.at[didx_b], sem_b).wait()
        plsc.subcore_barrier()

        def make_rdout(out_hbm):
            def rdout(r, c):
                pltpu.sync_copy(acc_sh.at[pl.ds(sid * _ZS + r * _ZR, _ZR)],
                                zrow_v)
                pltpu.sync_copy(
                    zrow_v, out_hbm.at[pl.ds(sid * _ZS + r * _ZR, _ZR)])
                return c
            return rdout

        def rem_out(out_hbm):
            pltpu.sync_copy(acc_sh.at[pl.ds(_NS * _ZS, 16)],
                            zrow_v.at[pl.ds(0, 16)])
            pltpu.sync_copy(zrow_v.at[pl.ds(0, 16)],
                            out_hbm.at[pl.ds(_NS * _ZS, 16)])

        @pl.when(cid == 0)
        def _():
            lax.fori_loop(0, _ZS // _ZR, make_rdout(out_lo_hbm), 0)

            @pl.when(sid == 0)
            def _():
                rem_out(out_lo_hbm)

        @pl.when(cid == 1)
        def _():
            lax.fori_loop(0, _ZS // _ZR, make_rdout(out_hi_hbm), 0)

            @pl.when(sid == 0)
            def _():
                rem_out(out_hi_hbm)

    return body


def _egather_body(h3_hbm, src_hbm, dst_hbm, sf_hbm, df_hbm,
                  idx_a, rows_a, idx_b, rows_b, sem_a, sem_b):
    cid = lax.axis_index("c")
    sid = lax.axis_index("s")
    wid = sid * _NC + cid

    def step(s, c):
        k = s // 2
        base = wid * _EPW + k * _CEG

        def do(eidx_hbm, out_hbm, idx_v, rows_v, sem):
            @pl.when(s >= 2)
            def _():
                pltpu.make_async_copy(
                    rows_v, out_hbm.at[pl.ds(base, _CEG)], sem).wait()

            pltpu.sync_copy(eidx_hbm.at[pl.ds(base, _CEG)], idx_v)
            pltpu.sync_copy(h3_hbm.at[idx_v], rows_v)
            pltpu.async_copy(rows_v, out_hbm.at[pl.ds(base, _CEG)], sem)

        @pl.when(s % 2 == 0)
        def _():
            do(src_hbm, sf_hbm, idx_a, rows_a, sem_a)

        @pl.when(s % 2 == 1)
        def _():
            do(dst_hbm, df_hbm, idx_b, rows_b, sem_b)

        return c

    lax.fori_loop(0, 2 * (_EPW // _CEG), step, 0)
    pltpu.make_async_copy(rows_a, sf_hbm.at[pl.ds(0, _CEG)], sem_a).wait()
    pltpu.make_async_copy(rows_b, df_hbm.at[pl.ds(0, _CEG)], sem_b).wait()


_BN = 1000
_BE = 4000


def _pre_body(x_ref, dis_ref, w_ref, olo_ref, ohi_ref):
    xb = jnp.nan_to_num(x_ref[...])
    res = (jnp.dot(xb, w_ref[...], preferred_element_type=jnp.float32)
           * dis_ref[...])
    hh = res.shape[1] // 2
    olo_ref[...] = res[:, :hh]
    ohi_ref[...] = res[:, hh:]


def _mid_body(slo_ref, shi_ref, plo_ref, phi_ref, dis_ref, b_ref, g_ref,
              bt_ref, m_ref, v_ref, w_ref, olo_ref, ohi_ref):
    dis = dis_ref[...]
    y = jnp.concatenate(
        [slo_ref[...] + plo_ref[...], shi_ref[...] + phi_ref[...]], axis=1)
    y = y * dis + b_ref[...]
    t = (y - m_ref[...]) * lax.rsqrt(v_ref[...] + _EPS) * g_ref[...] + bt_ref[...]
    t = jnp.maximum(t, 0.0)
    res = (jnp.dot(t, w_ref[...], preferred_element_type=jnp.float32) * dis)
    hh = res.shape[1] // 2
    olo_ref[...] = res[:, :hh]
    ohi_ref[...] = res[:, hh:]


def _fin_body(slo_ref, shi_ref, plo_ref, phi_ref, dis_ref, b_ref, g_ref,
              bt_ref, m_ref, v_ref, o_ref):
    dis = dis_ref[...]
    y = jnp.concatenate(
        [slo_ref[...] + plo_ref[...], shi_ref[...] + phi_ref[...]], axis=1)
    y = y * dis + b_ref[...]
    t = (y - m_ref[...]) * lax.rsqrt(v_ref[...] + _EPS) * g_ref[...] + bt_ref[...]
    o_ref[...] = jnp.maximum(t, 0.0)


def _mlp_body(sf_ref, df_ref, ea_ref, w1_ref, b1_ref, w2_ref, b2_ref,
              w3_ref, b3_ref, o_ref):
    ea = jnp.nan_to_num(ea_ref[...])
    ef = jnp.concatenate([sf_ref[...], df_ref[...], ea], axis=1)
    z = jnp.dot(ef, w1_ref[...], preferred_element_type=jnp.float32) + b1_ref[...]
    z = jnp.maximum(z, 0.0)
    z = jnp.maximum(jnp.dot(z, w2_ref[...], preferred_element_type=jnp.float32)
                    + b2_ref[...], 0.0)
    o_ref[...] = jnp.nan_to_num(
        jnp.dot(z, w3_ref[...], preferred_element_type=jnp.float32) + b3_ref[...])


def _row_spec(hd):
    return pl.BlockSpec((_BN, hd), lambda i: (i, 0))


def _full_spec(r, c):
    return pl.BlockSpec((r, c), lambda i: (0, 0))


def kernel(x, edge_index, edge_attr, W1, b1, W2, b2, W3, b3,
           g1, bt1, m1, v1, g2, bt2, m2, v2, g3, bt3, m3, v3,
           eW1, eb1, eW2, eb2, eW3, eb3):
    src = edge_index[0]
    dst = edge_index[1]
    mesh = plsc.VectorSubcoreMesh(core_axis_name="c", subcore_axis_name="s")

    degp = pl.kernel(
        _deg_body,
        out_type=jax.ShapeDtypeStruct((_NC * _N,), jnp.float32),
        mesh=mesh,
        compiler_params=pltpu.CompilerParams(use_tc_tiling_on_sc=False),
        scratch_types=[
            pltpu.VMEM((_CDEG,), jnp.int32),
            pltpu.VMEM((_CDEG,), jnp.float32),
            pltpu.VMEM((_ZS,), jnp.float32),
            pltpu.VMEM_SHARED((_N,), jnp.float32),
        ],
    )(dst)
    dis_col = lax.rsqrt(degp[:_N] + degp[_N:] + 1.0)[:, None]

    def prop(hp_lo, hp_hi, hdh, chp):
        return pl.kernel(
            _make_prop_body(hdh, chp),
            out_type=[jax.ShapeDtypeStruct((_N, hdh), jnp.float32),
                      jax.ShapeDtypeStruct((_N, hdh), jnp.float32)],
            mesh=mesh,
            compiler_params=pltpu.CompilerParams(use_tc_tiling_on_sc=False),
            scratch_types=[
                pltpu.VMEM((chp,), jnp.int32),
                pltpu.VMEM((chp,), jnp.int32),
                pltpu.VMEM((chp, hdh), jnp.float32),
                pltpu.VMEM((chp,), jnp.int32),
                pltpu.VMEM((chp,), jnp.int32),
                pltpu.VMEM((chp, hdh), jnp.float32),
                pltpu.VMEM((_ZR, hdh), jnp.float32),
                pltpu.VMEM_SHARED((_N, hdh), jnp.float32),
                pltpu.SemaphoreType.DMA,
                pltpu.SemaphoreType.DMA,
            ],
        )(hp_lo, hp_hi, src, dst)

    def half_outs(hd):
        return ([_row_spec(hd // 2), _row_spec(hd // 2)],
                [jax.ShapeDtypeStruct((_N, hd // 2), jnp.float32),
                 jax.ShapeDtypeStruct((_N, hd // 2), jnp.float32)])

    o_specs, o_shapes = half_outs(_H)
    p1_lo, p1_hi = pl.pallas_call(
        _pre_body,
        grid=(_N // _BN,),
        in_specs=[_row_spec(_D), _row_spec(1), _full_spec(_D, _H)],
        out_specs=o_specs,
        out_shape=o_shapes,
    )(x, dis_col, W1)

    s1_lo, s1_hi = prop(p1_lo, p1_hi, _H // 2, 400)

    def mid(s_lo, s_hi, p_lo, p_hi, b, g, bt, m, v, W, hin, hout):
        o_specs, o_shapes = half_outs(hout)
        return pl.pallas_call(
            _mid_body,
            grid=(_N // _BN,),
            in_specs=[_row_spec(hin // 2), _row_spec(hin // 2),
                      _row_spec(hin // 2), _row_spec(hin // 2),
                      _row_spec(1),
                      _full_spec(1, hin), _full_spec(1, hin),
                      _full_spec(1, hin), _full_spec(1, hin),
                      _full_spec(1, hin), _full_spec(hin, hout)],
            out_specs=o_specs,
            out_shape=o_shapes,
        )(s_lo, s_hi, p_lo, p_hi, dis_col, b.reshape(1, -1), g.reshape(1, -1),
          bt.reshape(1, -1), m.reshape(1, -1), v.reshape(1, -1), W)

    p2_lo, p2_hi = mid(s1_lo, s1_hi, p1_lo, p1_hi, b1, g1, bt1, m1, v1,
                       W2, _H, _H // 2)
    s2_lo, s2_hi = prop(p2_lo, p2_hi, _H // 4, 1000)
    p3_lo, p3_hi = mid(s2_lo, s2_hi, p2_lo, p2_hi, b2, g2, bt2, m2, v2,
                       W3, _H // 2, _H // 4)
    s3_lo, s3_hi = prop(p3_lo, p3_hi, _H // 8, 2000)

    h3 = pl.pallas_call(
        _fin_body,
        grid=(_N // _BN,),
        in_specs=[_row_spec(_H // 8), _row_spec(_H // 8),
                  _row_spec(_H // 8), _row_spec(_H // 8),
                  _row_spec(1),
                  _full_spec(1, _H // 4), _full_spec(1, _H // 4),
                  _full_spec(1, _H // 4), _full_spec(1, _H // 4),
                  _full_spec(1, _H // 4)],
        out_specs=_row_spec(_H // 4),
        out_shape=jax.ShapeDtypeStruct((_N, _H // 4), jnp.float32),
    )(s3_lo, s3_hi, p3_lo, p3_hi, dis_col, b3.reshape(1, -1),
      g3.reshape(1, -1), bt3.reshape(1, -1), m3.reshape(1, -1),
      v3.reshape(1, -1))

    sf, df = pl.kernel(
        _egather_body,
        out_type=[jax.ShapeDtypeStruct((_E, _H // 4), jnp.float32),
                  jax.ShapeDtypeStruct((_E, _H // 4), jnp.float32)],
        mesh=mesh,
        compiler_params=pltpu.CompilerParams(use_tc_tiling_on_sc=False),
        scratch_types=[
            pltpu.VMEM((_CEG,), jnp.int32),
            pltpu.VMEM((_CEG, _H // 4), jnp.float32),
            pltpu.VMEM((_CEG,), jnp.int32),
            pltpu.VMEM((_CEG, _H // 4), jnp.float32),
            pltpu.SemaphoreType.DMA,
            pltpu.SemaphoreType.DMA,
        ],
    )(h3, src, dst)

    er = pl.BlockSpec((_BE, _H // 4), lambda i: (i, 0))
    out = pl.pallas_call(
        _mlp_body,
        grid=(_E // _BE,),
        in_specs=[er, er, pl.BlockSpec((_BE, _DE), lambda i: (i, 0)),
                  _full_spec(2 * (_H // 4) + _DE, _H // 2),
                  _full_spec(1, _H // 2),
                  _full_spec(_H // 2, _H // 4), _full_spec(1, _H // 4),
                  _full_spec(_H // 4, 2), _full_spec(1, 2)],
        out_specs=pl.BlockSpec((_BE, 2), lambda i: (i, 0)),
        out_shape=jax.ShapeDtypeStruct((_E, 2), jnp.float32),
    )(sf, df, edge_attr, eW1, eb1.reshape(1, -1), eW2, eb2.reshape(1, -1),
      eW3, eb3.reshape(1, -1))
    return out
